# baseline (speedup 1.0000x reference)
import jax
import jax.numpy as jnp
from jax.experimental import pallas as pl
from jax.experimental.pallas import tpu as pltpu

_U, _N, _D, _O, _B = 16, 512, 256, 256, 32
_PREC = jax.lax.Precision.DEFAULT
_UPB = 8


def _res_step_kernel(X_ref, st_ref, W_ref, Win_ref, bias_ref,
                     srb_ref, Wout_ref, alr_ref, temp_ref,
                     ns_ref, out_ref, lr_ref):
    u = pl.program_id(0)

    @pl.when(u == 0)
    def _compute_lr():
        X = X_ref[...]
        alr = alr_ref[...]
        logits = jnp.sum(X * alr[None, :, :], axis=2)
        logits = logits / temp_ref[...]
        m = jnp.max(logits, axis=1, keepdims=True)
        e = jnp.exp(logits - m)
        lr_ref[...] = e / jnp.sum(e, axis=1, keepdims=True)

    for i in range(_UPB):
        uu = u * _UPB + i
        onehot = jax.lax.broadcasted_iota(jnp.int32, (_B, _U), 1) == uu
        lr_u = jnp.sum(jnp.where(onehot, lr_ref[...], 0.0), axis=1,
                       keepdims=True)

        x_u = X_ref[:, uu, :]
        st_u = st_ref[:, uu, :]
        feed = jnp.dot(x_u, Win_ref[i], precision=_PREC)
        echo = (jnp.dot(st_u, W_ref[i], precision=_PREC) * srb_ref[i]
                + bias_ref[i])
        ns = (1.0 - lr_u) * st_u + lr_u * jnp.tanh(feed + echo)
        ns_ref[:, uu, :] = ns
        out_ref[:, uu, :] = jnp.dot(ns, Wout_ref[i], precision=_PREC)


def kernel(X, state, W, Win, bias, Wout, sr, adaptive_lr, temperature,
           w_pos_u, w_pos_o, w_pos_d, win_pos_u, win_pos_o, win_pos_d):
    srb = jnp.broadcast_to(sr, (_U, 1, _N))
    alr2 = adaptive_lr[:, :, 0]
    temp2 = temperature.reshape(1, 1)

    new_state, output = pl.pallas_call(
        _res_step_kernel,
        grid=(_U // _UPB,),
        in_specs=[
            pl.BlockSpec((_B, _U, _D), lambda u: (0, 0, 0)),
            pl.BlockSpec((_B, _U, _N), lambda u: (0, 0, 0)),
            pl.BlockSpec((_UPB, _N, _N), lambda u: (u, 0, 0)),
            pl.BlockSpec((_UPB, _D, _N), lambda u: (u, 0, 0)),
            pl.BlockSpec((_UPB, 1, _N), lambda u: (u, 0, 0)),
            pl.BlockSpec((_UPB, 1, _N), lambda u: (u, 0, 0)),
            pl.BlockSpec((_UPB, _N, _O), lambda u: (u, 0, 0)),
            pl.BlockSpec((_U, _D), lambda u: (0, 0)),
            pl.BlockSpec((1, 1), lambda u: (0, 0)),
        ],
        out_specs=[
            pl.BlockSpec((_B, _U, _N), lambda u: (0, 0, 0)),
            pl.BlockSpec((_B, _U, _O), lambda u: (0, 0, 0)),
        ],
        out_shape=[
            jax.ShapeDtypeStruct((_B, _U, _N), jnp.float32),
            jax.ShapeDtypeStruct((_B, _U, _O), jnp.float32),
        ],
        scratch_shapes=[pltpu.VMEM((_B, _U), jnp.float32)],
    )(X, state, W, Win, bias, srb, Wout, alr2, temp2)

    return (new_state, output)


# speedup vs baseline: 1.0049x; 1.0049x over previous
import jax
import jax.numpy as jnp
from jax.experimental import pallas as pl
from jax.experimental.pallas import tpu as pltpu

_U, _N, _D, _O, _B = 16, 512, 256, 256, 32
_PREC = jax.lax.Precision.DEFAULT
_UPB = 4


def _res_step_kernel(X_ref, st_ref, W_ref, Win_ref, bias_ref,
                     srb_ref, Wout_ref, alr_ref, temp_ref,
                     ns_ref, out_ref, lr_ref):
    u = pl.program_id(0)

    @pl.when(u == 0)
    def _compute_lr():
        X = X_ref[...]
        alr = alr_ref[...]
        logits = jnp.sum(X * alr[None, :, :], axis=2)
        logits = logits / temp_ref[...]
        m = jnp.max(logits, axis=1, keepdims=True)
        e = jnp.exp(logits - m)
        lr_ref[...] = e / jnp.sum(e, axis=1, keepdims=True)

    for i in range(_UPB):
        uu = u * _UPB + i
        onehot = jax.lax.broadcasted_iota(jnp.int32, (_B, _U), 1) == uu
        lr_u = jnp.sum(jnp.where(onehot, lr_ref[...], 0.0), axis=1,
                       keepdims=True)

        x_u = X_ref[:, uu, :]
        st_u = st_ref[:, uu, :]
        feed = jnp.dot(x_u, Win_ref[i], precision=_PREC)
        echo = (jnp.dot(st_u, W_ref[i], precision=_PREC) * srb_ref[i]
                + bias_ref[i])
        ns = (1.0 - lr_u) * st_u + lr_u * jnp.tanh(feed + echo)
        ns_ref[:, uu, :] = ns
        out_ref[:, uu, :] = jnp.dot(ns, Wout_ref[i], precision=_PREC)


def kernel(X, state, W, Win, bias, Wout, sr, adaptive_lr, temperature,
           w_pos_u, w_pos_o, w_pos_d, win_pos_u, win_pos_o, win_pos_d):
    srb = jnp.broadcast_to(sr, (_U, 1, _N))
    alr2 = adaptive_lr[:, :, 0]
    temp2 = temperature.reshape(1, 1)

    new_state, output = pl.pallas_call(
        _res_step_kernel,
        grid=(_U // _UPB,),
        in_specs=[
            pl.BlockSpec((_B, _U, _D), lambda u: (0, 0, 0)),
            pl.BlockSpec((_B, _U, _N), lambda u: (0, 0, 0)),
            pl.BlockSpec((_UPB, _N, _N), lambda u: (u, 0, 0)),
            pl.BlockSpec((_UPB, _D, _N), lambda u: (u, 0, 0)),
            pl.BlockSpec((_UPB, 1, _N), lambda u: (u, 0, 0)),
            pl.BlockSpec((_UPB, 1, _N), lambda u: (u, 0, 0)),
            pl.BlockSpec((_UPB, _N, _O), lambda u: (u, 0, 0)),
            pl.BlockSpec((_U, _D), lambda u: (0, 0)),
            pl.BlockSpec((1, 1), lambda u: (0, 0)),
        ],
        out_specs=[
            pl.BlockSpec((_B, _U, _N), lambda u: (0, 0, 0)),
            pl.BlockSpec((_B, _U, _O), lambda u: (0, 0, 0)),
        ],
        out_shape=[
            jax.ShapeDtypeStruct((_B, _U, _N), jnp.float32),
            jax.ShapeDtypeStruct((_B, _U, _O), jnp.float32),
        ],
        scratch_shapes=[pltpu.VMEM((_B, _U), jnp.float32)],
    )(X, state, W, Win, bias, srb, Wout, alr2, temp2)

    return (new_state, output)
